# baseline (speedup 1.0000x reference)
import functools

import jax
import jax.numpy as jnp
from jax import lax
from jax.experimental import pallas as pl
from jax.experimental.pallas import tpu as pltpu
from jax.experimental.pallas import tpu_sc as plsc

N_EXP = 16
N_TOK = 2048
D_MODEL = 1024
D_FF = 512
TILE = 256
NT = N_TOK // TILE
NW = 32
EPT = (N_TOK * 2) // NW
TPW = N_TOK // NW


def _router_body(x_ref, wgt_ref, e1_ref, e2_ref, r1_ref, r2_ref,
                 p1_ref, p2_ref, cnt_ref):
    x = x_ref[...]
    logits = jnp.dot(x, wgt_ref[...], preferred_element_type=jnp.float32)
    m = jnp.max(logits, axis=-1, keepdims=True)
    p = jnp.exp(logits - m)
    p = p / jnp.sum(p, axis=-1, keepdims=True)
    idx = lax.broadcasted_iota(jnp.int32, p.shape, 1)
    big = jnp.int32(N_EXP + 1)
    m1 = jnp.max(p, axis=-1, keepdims=True)
    i1 = jnp.min(jnp.where(p >= m1, idx, big), axis=-1, keepdims=True)
    pm = jnp.where(idx == i1, -jnp.inf, p)
    m2 = jnp.max(pm, axis=-1, keepdims=True)
    i2 = jnp.min(jnp.where(pm >= m2, idx, big), axis=-1, keepdims=True)

    oh1 = (idx == i1).astype(jnp.float32)
    oh2 = (idx == i2).astype(jnp.float32)
    oh = oh1 + oh2
    s = oh
    sh = 1
    while sh < N_TOK:
        s = s + jnp.concatenate(
            [jnp.zeros((sh, N_EXP), jnp.float32), s[:-sh, :]], axis=0)
        sh *= 2
    excl = s - oh
    r1 = jnp.sum(excl * oh1, axis=-1, keepdims=True)
    r2 = jnp.sum((excl + oh1) * oh2, axis=-1, keepdims=True)

    e1_ref[...] = i1
    e2_ref[...] = i2
    r1_ref[...] = r1.astype(jnp.int32)
    r2_ref[...] = r2.astype(jnp.int32)
    p1_ref[...] = m1
    p2_ref[...] = m2
    cnt_ref[...] = jnp.sum(oh, axis=0, keepdims=True).astype(jnp.int32)


def _router(x_flat, WgT):
    i32 = jnp.int32
    f32 = jnp.float32
    outs = pl.pallas_call(
        _router_body,
        out_shape=[
            jax.ShapeDtypeStruct((N_TOK, 1), i32),
            jax.ShapeDtypeStruct((N_TOK, 1), i32),
            jax.ShapeDtypeStruct((N_TOK, 1), i32),
            jax.ShapeDtypeStruct((N_TOK, 1), i32),
            jax.ShapeDtypeStruct((N_TOK, 1), f32),
            jax.ShapeDtypeStruct((N_TOK, 1), f32),
            jax.ShapeDtypeStruct((1, N_EXP), i32),
        ],
    )(x_flat, WgT)
    return outs


def _dispatch_sc(x_flat, ek, rk):
    mesh = plsc.VectorSubcoreMesh(core_axis_name="c", subcore_axis_name="s")

    @functools.partial(
        pl.kernel,
        mesh=mesh,
        out_type=jax.ShapeDtypeStruct((N_EXP * N_TOK, D_MODEL // 2), jnp.float32),
        scratch_types=[
            pltpu.VMEM((EPT,), jnp.int32),
            pltpu.VMEM((EPT,), jnp.int32),
            pltpu.VMEM((EPT // 2,), jnp.int32),
            pltpu.VMEM((EPT // 2,), jnp.int32),
            pltpu.VMEM((EPT // 2, D_MODEL // 2), jnp.float32),
        ],
    )
    def k(x_hbm, ek_hbm, rk_hbm, xg_hbm, ev, rv, dva, dvb, rows):
        wid = lax.axis_index("s") * 2 + lax.axis_index("c")
        kk = wid & 1
        mm = wid >> 1
        base = mm * EPT
        pltpu.sync_copy(ek_hbm.at[kk, pl.ds(base, EPT)], ev)
        pltpu.sync_copy(rk_hbm.at[kk, pl.ds(base, EPT)], rv)
        for c in range(EPT // 16):
            e16 = ev[pl.ds(c * 16, 16)]
            r16 = rv[pl.ds(c * 16, 16)]
            d16 = e16 * N_TOK + r16
            half = c // (EPT // 32)
            off = (c % (EPT // 32)) * 16
            if half == 0:
                dva[pl.ds(off, 16)] = d16
            else:
                dvb[pl.ds(off, 16)] = d16
        pltpu.sync_copy(x_hbm.at[pl.ds(base, EPT // 2)], rows)
        pltpu.sync_copy(rows, xg_hbm.at[dva])
        pltpu.sync_copy(x_hbm.at[pl.ds(base + EPT // 2, EPT // 2)], rows)
        pltpu.sync_copy(rows, xg_hbm.at[dvb])

    return k(x_flat, ek, rk)


def _gmm_body(cnt_ref, xg_ref, w1_ref, w3_ref, w2_ref, yg_ref):
    e = pl.program_id(0)
    t = pl.program_id(1)

    @pl.when(t * TILE < cnt_ref[e])
    def _compute():
        xg = xg_ref[...]
        h1 = jnp.dot(xg, w1_ref[0], preferred_element_type=jnp.float32)
        h3 = jnp.dot(xg, w3_ref[0], preferred_element_type=jnp.float32)
        h = ((h1 / (1.0 + jnp.exp(-h1))) * h3).astype(jnp.bfloat16)
        yg_ref[...] = jnp.dot(h, w2_ref[0], preferred_element_type=jnp.float32)


def _row_blk(e, t, cnt):
    mt = (cnt[e] + TILE - 1) // TILE
    tc = jnp.minimum(t, jnp.maximum(mt - 1, 0))
    return e * NT + tc, 0


def _gmm(counts, xg, W1, W3, W2):
    grid_spec = pltpu.PrefetchScalarGridSpec(
        num_scalar_prefetch=1,
        grid=(N_EXP, NT),
        in_specs=[
            pl.BlockSpec((TILE, D_MODEL), _row_blk),
            pl.BlockSpec((1, D_MODEL, D_FF), lambda e, t, cnt: (e, 0, 0)),
            pl.BlockSpec((1, D_MODEL, D_FF), lambda e, t, cnt: (e, 0, 0)),
            pl.BlockSpec((1, D_FF, D_MODEL), lambda e, t, cnt: (e, 0, 0)),
        ],
        out_specs=pl.BlockSpec((TILE, D_MODEL), _row_blk),
    )
    return pl.pallas_call(
        _gmm_body,
        grid_spec=grid_spec,
        out_shape=jax.ShapeDtypeStruct((N_EXP * N_TOK, D_MODEL), jnp.float32),
        compiler_params=pltpu.CompilerParams(
            dimension_semantics=("arbitrary", "arbitrary"),
        ),
    )(counts, xg, W1, W3, W2)


def _gatherback_sc(yg, ek, rk):
    mesh = plsc.VectorSubcoreMesh(core_axis_name="c", subcore_axis_name="s")

    @functools.partial(
        pl.kernel,
        mesh=mesh,
        out_type=[
            jax.ShapeDtypeStruct((N_TOK, D_MODEL), jnp.float32),
            jax.ShapeDtypeStruct((N_TOK, D_MODEL), jnp.float32),
        ],
        scratch_types=[
            pltpu.VMEM((TPW,), jnp.int32),
            pltpu.VMEM((TPW,), jnp.int32),
            pltpu.VMEM((TPW,), jnp.int32),
            pltpu.VMEM((TPW,), jnp.int32),
            pltpu.VMEM((TPW // 2, D_MODEL), jnp.float32),
        ],
    )
    def k(yg_hbm, ek_hbm, rk_hbm, y1_hbm, y2_hbm,
          i1v, i2v, evs, rvs, rows):
        wid = lax.axis_index("s") * 2 + lax.axis_index("c")
        base = wid * TPW
        pltpu.sync_copy(ek_hbm.at[0, pl.ds(base, TPW)], evs)
        pltpu.sync_copy(rk_hbm.at[0, pl.ds(base, TPW)], rvs)
        for c in range(TPW // 16):
            i1v[pl.ds(c * 16, 16)] = (
                evs[pl.ds(c * 16, 16)] * N_TOK + rvs[pl.ds(c * 16, 16)])
        pltpu.sync_copy(ek_hbm.at[1, pl.ds(base, TPW)], evs)
        pltpu.sync_copy(rk_hbm.at[1, pl.ds(base, TPW)], rvs)
        for c in range(TPW // 16):
            i2v[pl.ds(c * 16, 16)] = (
                evs[pl.ds(c * 16, 16)] * N_TOK + rvs[pl.ds(c * 16, 16)])

        half = TPW // 2
        for c in range(2):
            pltpu.sync_copy(yg_hbm.at[i1v.at[pl.ds(c * half, half)]], rows)
            pltpu.sync_copy(rows, y1_hbm.at[pl.ds(base + c * half, half)])
            pltpu.sync_copy(yg_hbm.at[i2v.at[pl.ds(c * half, half)]], rows)
            pltpu.sync_copy(rows, y2_hbm.at[pl.ds(base + c * half, half)])

    return k(yg, ek, rk)


def _scale_body(y1_ref, y2_ref, p1_ref, p2_ref, y_ref):
    y_ref[...] = p1_ref[...] * y1_ref[...] + p2_ref[...] * y2_ref[...]


def _scale_add(y1, y2, p1, p2):
    blk = 256
    nb = N_TOK // blk
    return pl.pallas_call(
        _scale_body,
        grid=(nb,),
        in_specs=[
            pl.BlockSpec((blk, D_MODEL), lambda i: (i, 0)),
            pl.BlockSpec((blk, D_MODEL), lambda i: (i, 0)),
            pl.BlockSpec((blk, 1), lambda i: (i, 0)),
            pl.BlockSpec((blk, 1), lambda i: (i, 0)),
        ],
        out_specs=pl.BlockSpec((blk, D_MODEL), lambda i: (i, 0)),
        out_shape=jax.ShapeDtypeStruct((N_TOK, D_MODEL), jnp.float32),
    )(y1, y2, p1, p2)


def kernel(x, Wg, W1, W3, W2):
    Bs, Ts, C = x.shape
    x_flat = x.reshape(-1, C)

    e1, e2, r1, r2, p1, p2, cnt = _router(x_flat, Wg.T)
    ek = jnp.concatenate([e1.reshape(1, -1), e2.reshape(1, -1)], axis=0)
    rk = jnp.concatenate([r1.reshape(1, -1), r2.reshape(1, -1)], axis=0)
    counts = cnt.reshape(N_EXP)

    x_bf = x_flat.astype(jnp.bfloat16)
    x32 = lax.bitcast_convert_type(
        x_bf.reshape(N_TOK, D_MODEL // 2, 2), jnp.float32)
    xg32 = _dispatch_sc(x32, ek, rk)
    xg = lax.bitcast_convert_type(xg32, jnp.bfloat16).reshape(
        N_EXP * N_TOK, D_MODEL)
    yg = _gmm(counts, xg,
              W1.astype(jnp.bfloat16),
              W3.astype(jnp.bfloat16),
              W2.astype(jnp.bfloat16))
    y1, y2 = _gatherback_sc(yg, ek, rk)
    y = _scale_add(y1, y2, p1, p2)
    return y.reshape(Bs, Ts, C)


# speedup vs baseline: 3.9714x; 3.9714x over previous
import functools

import jax
import jax.numpy as jnp
from jax import lax
from jax.experimental import pallas as pl
from jax.experimental.pallas import tpu as pltpu
from jax.experimental.pallas import tpu_sc as plsc

N_EXP = 16
N_TOK = 2048
D_MODEL = 1024
D_FF = 512
TILE = 256
NT = N_TOK // TILE
NW = 32
EPT = (N_TOK * 2) // NW
TPW = N_TOK // NW


def _router_body(x_ref, wgt_ref, e1_ref, e2_ref, r1_ref, r2_ref,
                 p1_ref, p2_ref, cnt_ref):
    x = x_ref[...]
    logits = jnp.dot(x, wgt_ref[...], preferred_element_type=jnp.float32)
    m = jnp.max(logits, axis=-1, keepdims=True)
    p = jnp.exp(logits - m)
    p = p / jnp.sum(p, axis=-1, keepdims=True)
    idx = lax.broadcasted_iota(jnp.int32, p.shape, 1)
    big = jnp.int32(N_EXP + 1)
    m1 = jnp.max(p, axis=-1, keepdims=True)
    i1 = jnp.min(jnp.where(p >= m1, idx, big), axis=-1, keepdims=True)
    pm = jnp.where(idx == i1, -jnp.inf, p)
    m2 = jnp.max(pm, axis=-1, keepdims=True)
    i2 = jnp.min(jnp.where(pm >= m2, idx, big), axis=-1, keepdims=True)

    oh1 = (idx == i1).astype(jnp.float32)
    oh2 = (idx == i2).astype(jnp.float32)
    oh = oh1 + oh2
    s = oh
    sh = 1
    while sh < N_TOK:
        s = s + jnp.concatenate(
            [jnp.zeros((sh, N_EXP), jnp.float32), s[:-sh, :]], axis=0)
        sh *= 2
    excl = s - oh
    r1 = jnp.sum(excl * oh1, axis=-1, keepdims=True)
    r2 = jnp.sum((excl + oh1) * oh2, axis=-1, keepdims=True)

    e1_ref[...] = i1
    e2_ref[...] = i2
    r1_ref[...] = r1.astype(jnp.int32)
    r2_ref[...] = r2.astype(jnp.int32)
    p1_ref[...] = jnp.broadcast_to(m1, (N_TOK, 128))
    p2_ref[...] = jnp.broadcast_to(m2, (N_TOK, 128))
    cnt_ref[...] = jnp.sum(oh, axis=0, keepdims=True).astype(jnp.int32)


def _router(x_flat, WgT):
    i32 = jnp.int32
    f32 = jnp.float32
    outs = pl.pallas_call(
        _router_body,
        out_shape=[
            jax.ShapeDtypeStruct((N_TOK, 1), i32),
            jax.ShapeDtypeStruct((N_TOK, 1), i32),
            jax.ShapeDtypeStruct((N_TOK, 1), i32),
            jax.ShapeDtypeStruct((N_TOK, 1), i32),
            jax.ShapeDtypeStruct((N_TOK, 128), f32),
            jax.ShapeDtypeStruct((N_TOK, 128), f32),
            jax.ShapeDtypeStruct((1, N_EXP), i32),
        ],
    )(x_flat, WgT)
    return outs


def _dispatch_sc(x_flat, ek, rk, pk):
    mesh = plsc.VectorSubcoreMesh(core_axis_name="c", subcore_axis_name="s")

    @functools.partial(
        pl.kernel,
        mesh=mesh,
        out_type=[
            jax.ShapeDtypeStruct((N_EXP * N_TOK, D_MODEL), jnp.float32),
            jax.ShapeDtypeStruct((N_EXP * N_TOK, 128), jnp.float32),
        ],
        scratch_types=[
            pltpu.VMEM((EPT,), jnp.int32),
            pltpu.VMEM((EPT,), jnp.int32),
            pltpu.VMEM((EPT // 2,), jnp.int32),
            pltpu.VMEM((EPT // 2,), jnp.int32),
            pltpu.VMEM((EPT // 2, 128), jnp.float32),
            pltpu.VMEM((EPT // 2, D_MODEL), jnp.float32),
        ],
    )
    def k(x_hbm, ek_hbm, rk_hbm, pk_hbm, xg_hbm, pg_hbm,
          ev, rv, dva, dvb, prows, rows):
        wid = lax.axis_index("s") * 2 + lax.axis_index("c")
        kk = wid & 1
        mm = wid >> 1
        base = mm * EPT
        pltpu.sync_copy(ek_hbm.at[kk, pl.ds(base, EPT)], ev)
        pltpu.sync_copy(rk_hbm.at[kk, pl.ds(base, EPT)], rv)
        for c in range(EPT // 16):
            e16 = ev[pl.ds(c * 16, 16)]
            r16 = rv[pl.ds(c * 16, 16)]
            d16 = e16 * N_TOK + r16
            half = c // (EPT // 32)
            off = (c % (EPT // 32)) * 16
            if half == 0:
                dva[pl.ds(off, 16)] = d16
            else:
                dvb[pl.ds(off, 16)] = d16
        for h, dv in ((0, dva), (1, dvb)):
            hb = base + h * (EPT // 2)
            pltpu.sync_copy(pk_hbm.at[kk, pl.ds(hb, EPT // 2)], prows)
            pltpu.sync_copy(x_hbm.at[pl.ds(hb, EPT // 2)], rows)
            pltpu.sync_copy(rows, xg_hbm.at[dv])
            pltpu.sync_copy(prows, pg_hbm.at[dv])

    return k(x_flat, ek, rk, pk)


def _gmm_body(cnt_ref, xg_ref, pg_ref, w1_ref, w3_ref, w2_ref, yg_ref):
    e = pl.program_id(0)
    t = pl.program_id(1)

    @pl.when(t * TILE < cnt_ref[e])
    def _compute():
        xg = xg_ref[...]
        h1 = jnp.dot(xg, w1_ref[0], preferred_element_type=jnp.float32)
        h3 = jnp.dot(xg, w3_ref[0], preferred_element_type=jnp.float32)
        h = (h1 / (1.0 + jnp.exp(-h1))) * h3
        out = jnp.dot(h, w2_ref[0], preferred_element_type=jnp.float32)
        yg_ref[...] = out * pg_ref[:, 0:1]


def _row_blk(e, t, cnt):
    mt = (cnt[e] + TILE - 1) // TILE
    tc = jnp.minimum(t, jnp.maximum(mt - 1, 0))
    return e * NT + tc, 0


def _gmm(counts, xg, pg, W1, W3, W2):
    grid_spec = pltpu.PrefetchScalarGridSpec(
        num_scalar_prefetch=1,
        grid=(N_EXP, NT),
        in_specs=[
            pl.BlockSpec((TILE, D_MODEL), _row_blk),
            pl.BlockSpec((TILE, 128), _row_blk),
            pl.BlockSpec((1, D_MODEL, D_FF), lambda e, t, cnt: (e, 0, 0)),
            pl.BlockSpec((1, D_MODEL, D_FF), lambda e, t, cnt: (e, 0, 0)),
            pl.BlockSpec((1, D_FF, D_MODEL), lambda e, t, cnt: (e, 0, 0)),
        ],
        out_specs=pl.BlockSpec((TILE, D_MODEL), _row_blk),
    )
    return pl.pallas_call(
        _gmm_body,
        grid_spec=grid_spec,
        out_shape=jax.ShapeDtypeStruct((N_EXP * N_TOK, D_MODEL), jnp.float32),
        compiler_params=pltpu.CompilerParams(
            dimension_semantics=("arbitrary", "arbitrary"),
        ),
    )(counts, xg, pg, W1, W3, W2)


def _combine_sc(yg, ek, rk):
    mesh = plsc.VectorSubcoreMesh(core_axis_name="c", subcore_axis_name="s")

    @functools.partial(
        pl.kernel,
        mesh=mesh,
        out_type=jax.ShapeDtypeStruct((N_TOK, D_MODEL), jnp.float32),
        scratch_types=[
            pltpu.VMEM((TPW,), jnp.int32),
            pltpu.VMEM((TPW,), jnp.int32),
            pltpu.VMEM((TPW,), jnp.int32),
            pltpu.VMEM((TPW,), jnp.int32),
            pltpu.VMEM((TPW // 2, D_MODEL), jnp.float32),
            pltpu.VMEM((TPW // 2, D_MODEL), jnp.float32),
        ],
    )
    def k(yg_hbm, ek_hbm, rk_hbm, y_hbm,
          i1v, i2v, evs, rvs, rows1, rows2):
        wid = lax.axis_index("s") * 2 + lax.axis_index("c")
        base = wid * TPW
        pltpu.sync_copy(ek_hbm.at[0, pl.ds(base, TPW)], evs)
        pltpu.sync_copy(rk_hbm.at[0, pl.ds(base, TPW)], rvs)
        for c in range(TPW // 16):
            i1v[pl.ds(c * 16, 16)] = (
                evs[pl.ds(c * 16, 16)] * N_TOK + rvs[pl.ds(c * 16, 16)])
        pltpu.sync_copy(ek_hbm.at[1, pl.ds(base, TPW)], evs)
        pltpu.sync_copy(rk_hbm.at[1, pl.ds(base, TPW)], rvs)
        for c in range(TPW // 16):
            i2v[pl.ds(c * 16, 16)] = (
                evs[pl.ds(c * 16, 16)] * N_TOK + rvs[pl.ds(c * 16, 16)])

        half = TPW // 2
        for c in range(2):
            pltpu.sync_copy(yg_hbm.at[i1v.at[pl.ds(c * half, half)]], rows1)
            pltpu.sync_copy(yg_hbm.at[i2v.at[pl.ds(c * half, half)]], rows2)

            def row_loop(i, _):
                for cc in range(D_MODEL // 16):
                    a = rows1[i, pl.ds(cc * 16, 16)]
                    b = rows2[i, pl.ds(cc * 16, 16)]
                    rows1[i, pl.ds(cc * 16, 16)] = a + b
                return 0

            lax.fori_loop(0, half, row_loop, 0)
            pltpu.sync_copy(rows1, y_hbm.at[pl.ds(base + c * half, half)])

    return k(yg, ek, rk)


def kernel(x, Wg, W1, W3, W2):
    Bs, Ts, C = x.shape
    x_flat = x.reshape(-1, C)

    e1, e2, r1, r2, p1, p2, cnt = _router(x_flat, Wg.T)
    ek = jnp.concatenate([e1.reshape(1, -1), e2.reshape(1, -1)], axis=0)
    rk = jnp.concatenate([r1.reshape(1, -1), r2.reshape(1, -1)], axis=0)
    pk = jnp.concatenate(
        [p1.reshape(1, N_TOK, 128), p2.reshape(1, N_TOK, 128)], axis=0)
    counts = cnt.reshape(N_EXP)

    xg, pg = _dispatch_sc(x_flat, ek, rk, pk)
    yg = _gmm(counts, xg, pg, W1, W3, W2)
    y = _combine_sc(yg, ek, rk)
    return y.reshape(Bs, Ts, C)


# speedup vs baseline: 5.0925x; 1.2823x over previous
import functools

import jax
import jax.numpy as jnp
from jax import lax
from jax.experimental import pallas as pl
from jax.experimental.pallas import tpu as pltpu
from jax.experimental.pallas import tpu_sc as plsc

N_EXP = 16
N_TOK = 2048
D_MODEL = 1024
D_FF = 512
TILE = 256
NT = N_TOK // TILE
GMAX = (2 * N_TOK) // TILE + N_EXP
NW = 32
EPT = (N_TOK * 2) // NW
TPW = N_TOK // NW


def _router_body(x_ref, wgt_ref, e1_ref, e2_ref, r1_ref, r2_ref,
                 p1_ref, p2_ref, cnt_ref, te_ref, tt_ref, ntl_ref):
    x = x_ref[...]
    logits = jnp.dot(x, wgt_ref[...], preferred_element_type=jnp.float32)
    m = jnp.max(logits, axis=-1, keepdims=True)
    p = jnp.exp(logits - m)
    p = p / jnp.sum(p, axis=-1, keepdims=True)
    idx = lax.broadcasted_iota(jnp.int32, p.shape, 1)
    big = jnp.int32(N_EXP + 1)
    m1 = jnp.max(p, axis=-1, keepdims=True)
    i1 = jnp.min(jnp.where(p >= m1, idx, big), axis=-1, keepdims=True)
    pm = jnp.where(idx == i1, -jnp.inf, p)
    m2 = jnp.max(pm, axis=-1, keepdims=True)
    i2 = jnp.min(jnp.where(pm >= m2, idx, big), axis=-1, keepdims=True)

    oh1 = (idx == i1).astype(jnp.float32)
    oh2 = (idx == i2).astype(jnp.float32)
    oh = oh1 + oh2
    s = oh
    sh = 1
    while sh < N_TOK:
        s = s + jnp.concatenate(
            [jnp.zeros((sh, N_EXP), jnp.float32), s[:-sh, :]], axis=0)
        sh *= 2
    excl = s - oh
    r1 = jnp.sum(excl * oh1, axis=-1, keepdims=True)
    r2 = jnp.sum((excl + oh1) * oh2, axis=-1, keepdims=True)

    e1_ref[...] = i1
    e2_ref[...] = i2
    r1_ref[...] = r1.astype(jnp.int32)
    r2_ref[...] = r2.astype(jnp.int32)
    p1_ref[...] = jnp.broadcast_to(m1, (N_TOK, 128))
    p2_ref[...] = jnp.broadcast_to(m2, (N_TOK, 128))
    cntf = jnp.sum(oh, axis=0, keepdims=True)
    cnt_ref[...] = cntf.astype(jnp.int32)

    ntf = jnp.floor((cntf + (TILE - 1)) / TILE)
    r16 = lax.broadcasted_iota(jnp.int32, (N_EXP, N_EXP), 0)
    c16 = lax.broadcasted_iota(jnp.int32, (N_EXP, N_EXP), 1)
    upper = (r16 <= c16).astype(jnp.float32)
    incl = jnp.dot(ntf, upper, preferred_element_type=jnp.float32)
    excl = incl - ntf
    gi = lax.broadcasted_iota(jnp.int32, (GMAX, N_EXP), 0).astype(jnp.float32)
    exclB = jnp.broadcast_to(excl, (GMAX, N_EXP))
    inclB = jnp.broadcast_to(incl, (GMAX, N_EXP))
    ind = jnp.logical_and(gi >= exclB, gi < inclB).astype(jnp.float32)
    lane_e = lax.broadcasted_iota(
        jnp.int32, (GMAX, N_EXP), 1).astype(jnp.float32)
    te = jnp.sum(ind * lane_e, axis=-1, keepdims=True)
    tt = jnp.sum(ind * (gi - exclB), axis=-1, keepdims=True)
    tot = jnp.max(incl, axis=-1, keepdims=True)
    lane1 = lax.broadcasted_iota(jnp.int32, (1, N_EXP), 1)
    nt_last = jnp.sum(jnp.where(lane1 == (N_EXP - 1), ntf, 0.0),
                      axis=-1, keepdims=True)
    g1 = lax.broadcasted_iota(jnp.int32, (GMAX, 1), 0).astype(jnp.float32)
    act = g1 < tot
    te_ref[...] = jnp.where(act, te, float(N_EXP - 1)).astype(jnp.int32)
    tt_ref[...] = jnp.where(
        act, tt, jnp.maximum(nt_last - 1.0, 0.0)).astype(jnp.int32)
    ntl_ref[...] = tot.astype(jnp.int32)


def _router(x_flat, WgT):
    i32 = jnp.int32
    f32 = jnp.float32
    outs = pl.pallas_call(
        _router_body,
        out_shape=[
            jax.ShapeDtypeStruct((N_TOK, 1), i32),
            jax.ShapeDtypeStruct((N_TOK, 1), i32),
            jax.ShapeDtypeStruct((N_TOK, 1), i32),
            jax.ShapeDtypeStruct((N_TOK, 1), i32),
            jax.ShapeDtypeStruct((N_TOK, 128), f32),
            jax.ShapeDtypeStruct((N_TOK, 128), f32),
            jax.ShapeDtypeStruct((1, N_EXP), i32),
            jax.ShapeDtypeStruct((GMAX, 1), i32),
            jax.ShapeDtypeStruct((GMAX, 1), i32),
            jax.ShapeDtypeStruct((1, 1), i32),
        ],
    )(x_flat, WgT)
    return outs


def _dispatch_sc(x_flat, ek, rk, pk):
    mesh = plsc.VectorSubcoreMesh(core_axis_name="c", subcore_axis_name="s")

    @functools.partial(
        pl.kernel,
        mesh=mesh,
        out_type=[
            jax.ShapeDtypeStruct((N_EXP * N_TOK, D_MODEL), jnp.float32),
            jax.ShapeDtypeStruct((N_EXP * N_TOK, 128), jnp.float32),
        ],
        scratch_types=[
            pltpu.VMEM((EPT,), jnp.int32),
            pltpu.VMEM((EPT,), jnp.int32),
            pltpu.VMEM((EPT // 2,), jnp.int32),
            pltpu.VMEM((EPT // 2,), jnp.int32),
            pltpu.VMEM((EPT // 2, 128), jnp.float32),
            pltpu.VMEM((EPT // 2, D_MODEL), jnp.float32),
        ],
    )
    def k(x_hbm, ek_hbm, rk_hbm, pk_hbm, xg_hbm, pg_hbm,
          ev, rv, dva, dvb, prows, rows):
        wid = lax.axis_index("s") * 2 + lax.axis_index("c")
        kk = wid & 1
        mm = wid >> 1
        base = mm * EPT
        pltpu.sync_copy(ek_hbm.at[kk, pl.ds(base, EPT)], ev)
        pltpu.sync_copy(rk_hbm.at[kk, pl.ds(base, EPT)], rv)
        for c in range(EPT // 16):
            e16 = ev[pl.ds(c * 16, 16)]
            r16 = rv[pl.ds(c * 16, 16)]
            d16 = e16 * N_TOK + r16
            half = c // (EPT // 32)
            off = (c % (EPT // 32)) * 16
            if half == 0:
                dva[pl.ds(off, 16)] = d16
            else:
                dvb[pl.ds(off, 16)] = d16
        for h, dv in ((0, dva), (1, dvb)):
            hb = base + h * (EPT // 2)
            pltpu.sync_copy(pk_hbm.at[kk, pl.ds(hb, EPT // 2)], prows)
            pltpu.sync_copy(x_hbm.at[pl.ds(hb, EPT // 2)], rows)
            pltpu.sync_copy(rows, xg_hbm.at[dv])
            pltpu.sync_copy(prows, pg_hbm.at[dv])

    return k(x_flat, ek, rk, pk)


def _gmm_body(te_ref, tt_ref, ntl_ref, xg_ref, pg_ref,
              w1_ref, w3_ref, w2_ref, yg_ref):
    g = pl.program_id(0)

    @pl.when(g < ntl_ref[0])
    def _compute():
        xg = xg_ref[...]
        h1 = jnp.dot(xg, w1_ref[0], preferred_element_type=jnp.float32)
        h3 = jnp.dot(xg, w3_ref[0], preferred_element_type=jnp.float32)
        h = (h1 / (1.0 + jnp.exp(-h1))) * h3
        out = jnp.dot(h, w2_ref[0], preferred_element_type=jnp.float32)
        yg_ref[...] = out * pg_ref[:, 0:1]


def _row_blk(g, te, tt, ntl):
    return te[g] * NT + tt[g], 0


def _w_blk(g, te, tt, ntl):
    return te[g], 0, 0


def _gmm(te, tt, ntl, xg, pg, W1, W3, W2):
    grid_spec = pltpu.PrefetchScalarGridSpec(
        num_scalar_prefetch=3,
        grid=(GMAX,),
        in_specs=[
            pl.BlockSpec((TILE, D_MODEL), _row_blk),
            pl.BlockSpec((TILE, 128), _row_blk),
            pl.BlockSpec((1, D_MODEL, D_FF), _w_blk),
            pl.BlockSpec((1, D_MODEL, D_FF), _w_blk),
            pl.BlockSpec((1, D_FF, D_MODEL), _w_blk),
        ],
        out_specs=pl.BlockSpec((TILE, D_MODEL), _row_blk),
    )
    return pl.pallas_call(
        _gmm_body,
        grid_spec=grid_spec,
        out_shape=jax.ShapeDtypeStruct((N_EXP * N_TOK, D_MODEL), jnp.float32),
        compiler_params=pltpu.CompilerParams(
            dimension_semantics=("arbitrary",),
        ),
    )(te, tt, ntl, xg, pg, W1, W3, W2)


def _combine_sc(yg, ek, rk):
    mesh = plsc.VectorSubcoreMesh(core_axis_name="c", subcore_axis_name="s")

    @functools.partial(
        pl.kernel,
        mesh=mesh,
        out_type=jax.ShapeDtypeStruct((N_TOK, D_MODEL), jnp.float32),
        scratch_types=[
            pltpu.VMEM((TPW,), jnp.int32),
            pltpu.VMEM((TPW,), jnp.int32),
            pltpu.VMEM((TPW,), jnp.int32),
            pltpu.VMEM((TPW,), jnp.int32),
            pltpu.VMEM((TPW // 2, D_MODEL), jnp.float32),
            pltpu.VMEM((TPW // 2, D_MODEL), jnp.float32),
        ],
    )
    def k(yg_hbm, ek_hbm, rk_hbm, y_hbm,
          i1v, i2v, evs, rvs, rows1, rows2):
        wid = lax.axis_index("s") * 2 + lax.axis_index("c")
        base = wid * TPW
        pltpu.sync_copy(ek_hbm.at[0, pl.ds(base, TPW)], evs)
        pltpu.sync_copy(rk_hbm.at[0, pl.ds(base, TPW)], rvs)
        for c in range(TPW // 16):
            i1v[pl.ds(c * 16, 16)] = (
                evs[pl.ds(c * 16, 16)] * N_TOK + rvs[pl.ds(c * 16, 16)])
        pltpu.sync_copy(ek_hbm.at[1, pl.ds(base, TPW)], evs)
        pltpu.sync_copy(rk_hbm.at[1, pl.ds(base, TPW)], rvs)
        for c in range(TPW // 16):
            i2v[pl.ds(c * 16, 16)] = (
                evs[pl.ds(c * 16, 16)] * N_TOK + rvs[pl.ds(c * 16, 16)])

        half = TPW // 2
        for c in range(2):
            pltpu.sync_copy(yg_hbm.at[i1v.at[pl.ds(c * half, half)]], rows1)
            pltpu.sync_copy(yg_hbm.at[i2v.at[pl.ds(c * half, half)]], rows2)

            def row_loop(i, _):
                for cc in range(D_MODEL // 16):
                    a = rows1[i, pl.ds(cc * 16, 16)]
                    b = rows2[i, pl.ds(cc * 16, 16)]
                    rows1[i, pl.ds(cc * 16, 16)] = a + b
                return 0

            lax.fori_loop(0, half, row_loop, 0)
            pltpu.sync_copy(rows1, y_hbm.at[pl.ds(base + c * half, half)])

    return k(yg, ek, rk)


def kernel(x, Wg, W1, W3, W2):
    Bs, Ts, C = x.shape
    x_flat = x.reshape(-1, C)

    e1, e2, r1, r2, p1, p2, cnt, te, tt, ntl = _router(x_flat, Wg.T)
    ek = jnp.concatenate([e1.reshape(1, -1), e2.reshape(1, -1)], axis=0)
    rk = jnp.concatenate([r1.reshape(1, -1), r2.reshape(1, -1)], axis=0)
    pk = jnp.concatenate(
        [p1.reshape(1, N_TOK, 128), p2.reshape(1, N_TOK, 128)], axis=0)
    xg, pg = _dispatch_sc(x_flat, ek, rk, pk)
    yg = _gmm(te.reshape(GMAX), tt.reshape(GMAX), ntl.reshape(1),
              xg, pg, W1, W3, W2)
    y = _combine_sc(yg, ek, rk)
    return y.reshape(Bs, Ts, C)
